# baseline (device time: 23560 ns/iter reference)
import jax
import jax.numpy as jnp
from jax import lax
from jax.experimental import pallas as pl
from jax.experimental.pallas import tpu as pltpu

N_DEV = 16
N_IDX = 512
ROWS_PER = 2048
D = 256
N_RAILS = 4
Q = N_IDX // N_RAILS

PLANE_MASKS = (1, 2, 3)
Z_MASKS = (4, 8, 12)
ALL_MASKS = PLANE_MASKS + Z_MASKS


def kernel(table, idx):
    idx2 = idx.reshape(N_IDX, 1)

    def body(table_ref, idx_ref, out_ref, init_ref, piece_ref, recv_ref,
             send_sems, recv_sems):
        my = lax.axis_index("i")

        barrier_sem = pltpu.get_barrier_semaphore()
        for m in ALL_MASKS:
            pl.semaphore_signal(
                barrier_sem,
                inc=1,
                device_id=(my ^ m,),
                device_id_type=pl.DeviceIdType.MESH,
            )

        table_bf16 = table_ref[:, :].astype(jnp.bfloat16)

        def partial_quarter(r):
            local = idx_ref[pl.ds(r * Q, Q), :] - my * ROWS_PER
            cols = lax.broadcasted_iota(jnp.int16, (Q, ROWS_PER), 1)
            onehot = (cols == local.astype(jnp.int16)).astype(jnp.bfloat16)
            acc = lax.dot_general(
                onehot,
                table_bf16,
                (((1,), (0,)), ((), ())),
                preferred_element_type=jnp.float32,
            )
            init_ref[r] = acc.astype(jnp.bfloat16)

        def start(src_ref, slot, k, rail, mask):
            rdma = pltpu.make_async_remote_copy(
                src_ref=src_ref,
                dst_ref=recv_ref.at[slot, k, rail],
                send_sem=send_sems.at[slot, k, rail],
                recv_sem=recv_sems.at[slot, k, rail],
                device_id=(my ^ mask,),
                device_id_type=pl.DeviceIdType.MESH,
            )
            rdma.start()
            return rdma

        plane = {}
        partial_quarter(0)
        pl.semaphore_wait(barrier_sem, len(ALL_MASKS))
        for k, m in enumerate(PLANE_MASKS):
            plane[k, 0] = start(init_ref.at[0], 0, k, 0, m)
        for r in range(1, N_RAILS):
            partial_quarter(r)
            for k, m in enumerate(PLANE_MASKS):
                plane[k, r] = start(init_ref.at[r], 0, k, r, m)

        zline = {}
        for r in range(N_RAILS):
            for k in range(len(PLANE_MASKS)):
                plane[k, r].wait_recv()
            piece_ref[r] = (
                init_ref[r]
                + recv_ref[0, 0, r]
                + recv_ref[0, 1, r]
                + recv_ref[0, 2, r]
            )
            for k, m in enumerate(Z_MASKS):
                zline[k, r] = start(piece_ref.at[r], 1, k, r, m)

        for r in range(N_RAILS):
            for k in range(len(Z_MASKS)):
                zline[k, r].wait_recv()
            out_ref[pl.ds(r * Q, Q), :] = (
                piece_ref[r]
                + recv_ref[1, 0, r]
                + recv_ref[1, 1, r]
                + recv_ref[1, 2, r]
            )

        for d in list(plane.values()) + list(zline.values()):
            d.wait_send()

    return pl.pallas_call(
        body,
        out_shape=jax.ShapeDtypeStruct((N_IDX, D), jnp.bfloat16),
        in_specs=[
            pl.BlockSpec(memory_space=pltpu.VMEM),
            pl.BlockSpec(memory_space=pltpu.VMEM),
        ],
        out_specs=pl.BlockSpec(memory_space=pltpu.VMEM),
        scratch_shapes=[
            pltpu.VMEM((N_RAILS, Q, D), jnp.bfloat16),
            pltpu.VMEM((N_RAILS, Q, D), jnp.bfloat16),
            pltpu.VMEM((2, 3, N_RAILS, Q, D), jnp.bfloat16),
            pltpu.SemaphoreType.DMA((2, 3, N_RAILS)),
            pltpu.SemaphoreType.DMA((2, 3, N_RAILS)),
        ],
        compiler_params=pltpu.CompilerParams(collective_id=0),
    )(table, idx2)


# device time: 19406 ns/iter; 1.2141x vs baseline; 1.2141x over previous
import jax
import jax.numpy as jnp
from jax import lax
from jax.experimental import pallas as pl
from jax.experimental.pallas import tpu as pltpu

N_DEV = 16
N_STEPS = 4
N_IDX = 512
ROWS_PER = 2048
D = 256
N_RAILS = 4
Q = N_IDX // N_RAILS

BASE_MASKS = (1, 3, 4, 8)
MASKS = tuple(
    tuple(BASE_MASKS[(s + r) % N_STEPS] for s in range(N_STEPS))
    for r in range(N_RAILS)
)


def kernel(table, idx):
    idx2 = idx.reshape(N_IDX, 1)

    def body(table_ref, idx_ref, out_ref, acc_ref, recv_ref, send_sems,
             recv_sems):
        my = lax.axis_index("i")

        barrier_sem = pltpu.get_barrier_semaphore()
        for m in BASE_MASKS:
            pl.semaphore_signal(
                barrier_sem,
                inc=1,
                device_id=(my ^ m,),
                device_id_type=pl.DeviceIdType.MESH,
            )

        table_bf16 = table_ref[:, :].astype(jnp.bfloat16)

        def partial_quarter(r):
            local = idx_ref[pl.ds(r * Q, Q), :] - my * ROWS_PER
            cols = lax.broadcasted_iota(jnp.int16, (Q, ROWS_PER), 1)
            onehot = (cols == local.astype(jnp.int16)).astype(jnp.bfloat16)
            acc = lax.dot_general(
                onehot,
                table_bf16,
                (((1,), (0,)), ((), ())),
                preferred_element_type=jnp.float32,
            )
            acc_ref[r, 0] = acc.astype(jnp.bfloat16)

        def start(rail, s, src_pp):
            rdma = pltpu.make_async_remote_copy(
                src_ref=acc_ref.at[rail, src_pp],
                dst_ref=recv_ref.at[s, rail],
                send_sem=send_sems.at[s, rail],
                recv_sem=recv_sems.at[s, rail],
                device_id=(my ^ MASKS[rail][s],),
                device_id_type=pl.DeviceIdType.MESH,
            )
            rdma.start()
            return rdma

        sends = {r: [None] * N_STEPS for r in range(N_RAILS)}
        partial_quarter(0)
        pl.semaphore_wait(barrier_sem, N_STEPS)
        sends[0][0] = start(0, 0, 0)
        for r in range(1, N_RAILS):
            partial_quarter(r)
            sends[r][0] = start(r, 0, 0)

        for s in range(N_STEPS):
            pp, nxt = s % 2, (s + 1) % 2
            slow = (3 - s) % N_RAILS
            for rail in [r for r in range(N_RAILS) if r != slow] + [slow]:
                sends[rail][s].wait_recv()
                if s >= 1:
                    sends[rail][s - 1].wait_send()
                summed = acc_ref[rail, pp] + recv_ref[s, rail]
                if s + 1 < N_STEPS:
                    acc_ref[rail, nxt] = summed
                    sends[rail][s + 1] = start(rail, s + 1, nxt)
                else:
                    out_ref[pl.ds(rail * Q, Q), :] = summed

        for rail in range(N_RAILS):
            sends[rail][N_STEPS - 1].wait_send()

    return pl.pallas_call(
        body,
        out_shape=jax.ShapeDtypeStruct((N_IDX, D), jnp.bfloat16),
        in_specs=[
            pl.BlockSpec(memory_space=pltpu.VMEM),
            pl.BlockSpec(memory_space=pltpu.VMEM),
        ],
        out_specs=pl.BlockSpec(memory_space=pltpu.VMEM),
        scratch_shapes=[
            pltpu.VMEM((N_RAILS, 2, Q, D), jnp.bfloat16),
            pltpu.VMEM((N_STEPS, N_RAILS, Q, D), jnp.bfloat16),
            pltpu.SemaphoreType.DMA((N_STEPS, N_RAILS)),
            pltpu.SemaphoreType.DMA((N_STEPS, N_RAILS)),
        ],
        compiler_params=pltpu.CompilerParams(collective_id=0),
    )(table, idx2)
